# baseline (speedup 1.0000x reference)
import jax
import jax.numpy as jnp
from jax.experimental import pallas as pl
from jax.experimental.pallas import tpu as pltpu

B, C, H, W = 64, 384, 28, 28


def _copy_body(perm_ref, x_ref, o_ref):
    o_ref[...] = x_ref[...]


@jax.jit
def _permute(x, perm):
    grid_spec = pltpu.PrefetchScalarGridSpec(
        num_scalar_prefetch=1,
        grid=(C,),
        in_specs=[
            pl.BlockSpec((B, 1, H, W), lambda i, perm_ref: (0, perm_ref[i], 0, 0)),
        ],
        out_specs=pl.BlockSpec((B, 1, H, W), lambda i, perm_ref: (0, i, 0, 0)),
    )
    return pl.pallas_call(
        _copy_body,
        grid_spec=grid_spec,
        out_shape=jax.ShapeDtypeStruct((B, C, H, W), jnp.float32),
    )(perm, x)


def kernel(x, perm):
    y = _permute(x, perm)
    logdet = jnp.zeros((B,), dtype=x.dtype)
    return (y, logdet)


# speedup vs baseline: 1.8545x; 1.8545x over previous
import functools

import jax
import jax.numpy as jnp
from jax import lax
from jax.experimental import pallas as pl
from jax.experimental.pallas import tpu as pltpu, tpu_sc as plsc

B, C, H, W = 64, 384, 28, 28
NC, NS, L = 2, 16, 16
NW = NC * NS
BPW = B // NW
K = 64
NCHUNK = C // K


def _body(x_hbm, perm_hbm, out_hbm,
          perm_v, buf0, buf1, gs0, gs1, ws0, ws1):
    wid = lax.axis_index("s") * NC + lax.axis_index("c")

    pltpu.sync_copy(perm_hbm, perm_v)

    bufs = (buf0, buf1)
    gsems = (gs0, gs1)
    wsems = (ws0, ws1)

    def copy_in(b, i, s):
        return pltpu.async_copy(
            x_hbm.at[b].at[perm_v.at[pl.ds(i * K, K)]], bufs[s], gsems[s])

    def copy_out(b, i, s):
        return pltpu.async_copy(bufs[s], out_hbm.at[b, pl.ds(i * K, K)],
                                wsems[s])

    for bl in range(BPW):
        b = wid * BPW + bl
        g = {}
        w = {}
        g[0] = copy_in(b, 0, 0)
        for i in range(NCHUNK):
            s = i % 2
            if i + 1 < NCHUNK:
                if i - 1 >= 0:
                    w[i - 1].wait()
                g[i + 1] = copy_in(b, i + 1, (i + 1) % 2)
            g[i].wait()
            w[i] = copy_out(b, i, s)
        w[NCHUNK - 2].wait()
        w[NCHUNK - 1].wait()


@jax.jit
def _permute(x, perm):
    mesh = plsc.VectorSubcoreMesh(core_axis_name="c", subcore_axis_name="s")
    run = functools.partial(
        pl.kernel,
        mesh=mesh,
        compiler_params=pltpu.CompilerParams(use_tc_tiling_on_sc=False),
        out_type=jax.ShapeDtypeStruct((B, C, H * W), jnp.float32),
        scratch_types=[
            pltpu.VMEM((C,), jnp.int32),
            pltpu.VMEM((K, H * W), jnp.float32),
            pltpu.VMEM((K, H * W), jnp.float32),
            pltpu.SemaphoreType.DMA,
            pltpu.SemaphoreType.DMA,
            pltpu.SemaphoreType.DMA,
            pltpu.SemaphoreType.DMA,
        ],
    )(_body)
    return run(x.reshape(B, C, H * W), perm)


def kernel(x, perm):
    y = _permute(x, perm).reshape(B, C, H, W)
    logdet = jnp.zeros((B,), dtype=x.dtype)
    return (y, logdet)
